# Initial kernel scaffold; baseline (speedup 1.0000x reference)
import jax
import jax.numpy as jnp
from jax.experimental import pallas as pl


def kernel(x, edge_index, W1l, b1l, W1r, W2l, b2l, W2r):
    raise NotImplementedError("write your pallas kernel here")


# speedup vs baseline: 4.5055x; 4.5055x over previous
import jax
import jax.numpy as jnp
from jax import lax
from jax.experimental import pallas as pl
from jax.experimental.pallas import tpu as pltpu
from jax.experimental.pallas import tpu_sc as plsc

N = 10000
E = 320000
D = 128
N_PAD = 10240
D_AUG = 144
NW = 32
K = 128
C = -(-E // (NW * K))
E_PAD = NW * C * K
STRIPE = N_PAD // 16
BLK = 512


def _make_agg(d):
    mesh = plsc.VectorSubcoreMesh(core_axis_name="c", subcore_axis_name="s")

    def body(x_hbm, src_hbm, dst_hbm, z_hbm, out_hbm, src_v, dst_v, row_v, sem, acc):
        c = lax.axis_index("c")
        s = lax.axis_index("s")
        wid = c * 16 + s
        pltpu.sync_copy(src_hbm.at[wid], src_v)
        pltpu.sync_copy(dst_hbm.at[wid], dst_v)
        pltpu.sync_copy(z_hbm, acc.at[pl.ds(s * STRIPE, STRIPE)])
        plsc.subcore_barrier()

        def step(j, carry):
            pltpu.async_copy(x_hbm.at[src_v.at[j]], row_v, sem).wait()
            pltpu.sync_copy(row_v, acc.at[dst_v.at[j]], add=True)
            return carry

        lax.fori_loop(0, C, step, 0)
        plsc.subcore_barrier()
        pltpu.sync_copy(acc.at[pl.ds(s * STRIPE, STRIPE)],
                        out_hbm.at[pl.ds(c * N_PAD + s * STRIPE, STRIPE)])

    return pl.kernel(
        body,
        out_type=jax.ShapeDtypeStruct((2 * N_PAD, d), jnp.float32),
        mesh=mesh,
        compiler_params=pltpu.CompilerParams(use_tc_tiling_on_sc=False),
        scratch_types=[
            pltpu.VMEM((C, K), jnp.int32),
            pltpu.VMEM((C, K), jnp.int32),
            pltpu.VMEM((K, d), jnp.float32),
            pltpu.SemaphoreType.DMA,
            pltpu.VMEM_SHARED((N_PAD, d), jnp.float32),
        ],
    )


_agg_aug = _make_agg(D_AUG)
_agg_plain = _make_agg(D)


def _dense1_body(p0, p1, x_ref, wl, wr, b, h_ref, inv_ref):
    s = p0[...] + p1[...]
    deg = s[:, D:D + 1]
    inv = 1.0 / jnp.maximum(deg, 1.0)
    mean = s[:, :D] * inv
    h = (jnp.dot(mean, wl[...], preferred_element_type=jnp.float32)
         + jnp.dot(x_ref[...], wr[...], preferred_element_type=jnp.float32)
         + b[...])
    h_ref[...] = jnp.maximum(h, 0.0)
    inv_ref[...] = inv


_dense1 = pl.pallas_call(
    _dense1_body,
    grid=(N_PAD // BLK,),
    in_specs=[
        pl.BlockSpec((BLK, D_AUG), lambda i: (i, 0)),
        pl.BlockSpec((BLK, D_AUG), lambda i: (i, 0)),
        pl.BlockSpec((BLK, D), lambda i: (i, 0)),
        pl.BlockSpec((D, D), lambda i: (0, 0)),
        pl.BlockSpec((D, D), lambda i: (0, 0)),
        pl.BlockSpec((1, D), lambda i: (0, 0)),
    ],
    out_specs=[pl.BlockSpec((BLK, D), lambda i: (i, 0)),
               pl.BlockSpec((BLK, 1), lambda i: (i, 0))],
    out_shape=[jax.ShapeDtypeStruct((N_PAD, D), jnp.float32),
               jax.ShapeDtypeStruct((N_PAD, 1), jnp.float32)],
)


def _dense2_body(p0, p1, h_ref, inv_ref, wl, wr, b, out_ref):
    mean = (p0[...] + p1[...]) * inv_ref[...]
    out_ref[...] = (jnp.dot(mean, wl[...], preferred_element_type=jnp.float32)
                    + jnp.dot(h_ref[...], wr[...], preferred_element_type=jnp.float32)
                    + b[...])


_dense2 = pl.pallas_call(
    _dense2_body,
    grid=(N_PAD // BLK,),
    in_specs=[
        pl.BlockSpec((BLK, D), lambda i: (i, 0)),
        pl.BlockSpec((BLK, D), lambda i: (i, 0)),
        pl.BlockSpec((BLK, D), lambda i: (i, 0)),
        pl.BlockSpec((BLK, 1), lambda i: (i, 0)),
        pl.BlockSpec((D, D), lambda i: (0, 0)),
        pl.BlockSpec((D, D), lambda i: (0, 0)),
        pl.BlockSpec((1, D), lambda i: (0, 0)),
    ],
    out_specs=pl.BlockSpec((BLK, D), lambda i: (i, 0)),
    out_shape=jax.ShapeDtypeStruct((N_PAD, D), jnp.float32),
)


def kernel(x, edge_index, W1l, b1l, W1r, W2l, b2l, W2r):
    src = edge_index[0].astype(jnp.int32)
    dst = edge_index[1].astype(jnp.int32)
    src_t = jnp.concatenate([src, jnp.zeros((E_PAD - E,), jnp.int32)]).reshape(NW, C, K)
    dst_t = jnp.concatenate([dst, jnp.full((E_PAD - E,), N, jnp.int32)]).reshape(NW, C, K)

    x_aug = jnp.zeros((N_PAD, D_AUG), jnp.float32)
    x_aug = x_aug.at[:N, :D].set(x)
    x_aug = x_aug.at[:N, D].set(1.0)
    x_pad = x_aug[:, :D]
    z_aug = jnp.zeros((STRIPE, D_AUG), jnp.float32)
    z_plain = jnp.zeros((STRIPE, D), jnp.float32)

    p = _agg_aug(x_aug, src_t, dst_t, z_aug)
    h, inv = _dense1(p[:N_PAD], p[N_PAD:], x_pad, W1l.T, W1r.T, b1l[None, :])
    p2 = _agg_plain(h, src_t, dst_t, z_plain)
    out = _dense2(p2[:N_PAD], p2[N_PAD:], h, inv, W2l.T, W2r.T, b2l[None, :])
    return out[:N]
